# baseline (speedup 1.0000x reference)
import functools

import jax
import jax.numpy as jnp
from jax import lax
from jax.experimental import pallas as pl
from jax.experimental.pallas import tpu as pltpu
from jax.experimental.pallas import tpu_sc as plsc

B = 16384
MF = 64
HALF = 64
BD = 768
H1 = 64
CW = MF + H1
NROWS = 100000

NC = 2
NS = 16
NW = NC * NS
BPW = B // NW

_DOT = functools.partial(jnp.dot, preferred_element_type=jnp.float32,
                         precision=lax.Precision.DEFAULT)


RT = 4096


def _pre_body(gT_r, mT_r, tab_r):
    tab_r[...] = jnp.concatenate([gT_r[...], mT_r[...]], axis=0).T


def _tc_pre(gT, mT):
    col = lambda i: (0, i)
    row = lambda i: (i, 0)
    return pl.pallas_call(
        _pre_body,
        grid=(pl.cdiv(NROWS, RT),),
        in_specs=[
            pl.BlockSpec((MF, RT), col),
            pl.BlockSpec((HALF, RT), col),
        ],
        out_specs=pl.BlockSpec((RT, CW), row),
        out_shape=jax.ShapeDtypeStruct((NROWS, CW), jnp.float32),
    )(gT, mT)


def _sc_gather_bert(item_idx, bert):
    mesh = plsc.VectorSubcoreMesh(core_axis_name="c", subcore_axis_name="s")
    CH = 64
    NCH = BPW // CH

    @functools.partial(
        pl.kernel, mesh=mesh,
        out_type=jax.ShapeDtypeStruct((B, BD), jnp.float32),
        scratch_types=[
            pltpu.VMEM((BPW,), jnp.int32),
            pltpu.VMEM((CH, BD), jnp.float32),
            pltpu.SemaphoreType.DMA,
        ])
    def k(iidx_h, bert_h, obert_h, iidx_v, buf_v, sem):
        wid = lax.axis_index("s") * NC + lax.axis_index("c")
        base = wid * BPW
        pltpu.sync_copy(iidx_h.at[pl.ds(base, BPW)], iidx_v)

        @pl.loop(0, NCH)
        def _(ci):
            off = ci * CH
            ii = iidx_v.at[pl.ds(off, CH)]
            pltpu.async_copy(bert_h.at[ii], buf_v, sem).wait()
            pltpu.sync_copy(buf_v, obert_h.at[pl.ds(base + off, CH)])

    return k(item_idx, bert)


def _sc_gather_tab(idx, tab):
    mesh = plsc.VectorSubcoreMesh(core_axis_name="c", subcore_axis_name="s")
    CH = 128
    NCH = BPW // CH

    @functools.partial(
        pl.kernel, mesh=mesh,
        out_type=jax.ShapeDtypeStruct((B, CW), jnp.float32),
        scratch_types=[
            pltpu.VMEM((BPW,), jnp.int32),
            pltpu.VMEM((CH, CW), jnp.float32),
            pltpu.SemaphoreType.DMA,
        ])
    def k(idx_h, tab_h, o_h, idx_v, buf_v, sem):
        wid = lax.axis_index("s") * NC + lax.axis_index("c")
        base = wid * BPW
        pltpu.sync_copy(idx_h.at[pl.ds(base, BPW)], idx_v)

        @pl.loop(0, NCH)
        def _(ci):
            off = ci * CH
            pltpu.async_copy(tab_h.at[idx_v.at[pl.ds(off, CH)]], buf_v,
                             sem).wait()
            pltpu.sync_copy(buf_v, o_h.at[pl.ds(base + off, CH)])

    return k(idx, tab)


BT = 1024


def _tc_body(u_r, i_r, bt_r, w1u_r, w1i_r, w1b_r, b1_r, w2a_r, w2b_r, b2_r,
             o_r):
    u = u_r[...]
    it = i_r[...]
    h = (_DOT(u[:, MF:], w1u_r[...]) + _DOT(it[:, MF:], w1i_r[...])
         + _DOT(bt_r[...], w1b_r[...]) + b1_r[...])
    h = jnp.maximum(h, 0.0)
    g = u[:, :MF] * it[:, :MF]
    o = (jnp.sum(g * w2a_r[...], axis=1, keepdims=True)
         + jnp.sum(h * w2b_r[...], axis=1, keepdims=True) + b2_r[...])
    o_r[...] = o


def _tc_main(ug, ig, bertg, w1u, w1i, w1b, b1, W2, b2):
    b1r = b1.reshape(1, H1)
    w2a = W2[:MF, 0].reshape(1, MF)
    w2b = W2[MF:, 0].reshape(1, H1)
    b2r = b2.reshape(1, 1)

    row = lambda i: (i, 0)
    fixed = lambda i: (0, 0)
    return pl.pallas_call(
        _tc_body,
        grid=(B // BT,),
        in_specs=[
            pl.BlockSpec((BT, CW), row),
            pl.BlockSpec((BT, CW), row),
            pl.BlockSpec((BT, BD), row),
            pl.BlockSpec((HALF, H1), fixed),
            pl.BlockSpec((HALF, H1), fixed),
            pl.BlockSpec((BD, H1), fixed),
            pl.BlockSpec((1, H1), fixed),
            pl.BlockSpec((1, MF), fixed),
            pl.BlockSpec((1, H1), fixed),
            pl.BlockSpec((1, 1), fixed),
        ],
        out_specs=pl.BlockSpec((BT, 1), row),
        out_shape=jax.ShapeDtypeStruct((B, 1), jnp.float32),
    )(ug, ig, bertg, w1u, w1i, w1b, b1r, w2a, w2b, b2r)


def kernel(user_idx, item_idx, user_emb_gmf, item_emb_gmf, user_emb_mlp,
           item_emb_mlp, item_bert, W1, b1, W2, b2):
    user_idx = user_idx.astype(jnp.int32)
    item_idx = item_idx.astype(jnp.int32)
    w1u = W1[:HALF]
    w1i = W1[HALF:2 * HALF]
    w1b = W1[2 * HALF:]
    bertg = _sc_gather_bert(item_idx, item_bert)
    itab = _tc_pre(item_emb_gmf.T, item_emb_mlp.T)
    itab, bertg = lax.optimization_barrier((itab, bertg))
    itg = _sc_gather_tab(item_idx, itab)
    utab = _tc_pre(user_emb_gmf.T, user_emb_mlp.T)
    utg = _sc_gather_tab(user_idx, utab)
    return _tc_main(utg, itg, bertg, w1u, w1i, w1b, b1, W2, b2)


# speedup vs baseline: 1.1072x; 1.1072x over previous
import functools

import jax
import jax.numpy as jnp
from jax import lax
from jax.experimental import pallas as pl
from jax.experimental.pallas import tpu as pltpu
from jax.experimental.pallas import tpu_sc as plsc

B = 16384
MF = 64
HALF = 64
BD = 768
H1 = 64
CW = MF + H1
NROWS = 100000

NC = 2
NS = 16
NW = NC * NS
BPW = B // NW

_DOT = functools.partial(jnp.dot, preferred_element_type=jnp.float32,
                         precision=lax.Precision.DEFAULT)


RT = 4096


def _pre_body(ugT_r, umT_r, igT_r, imT_r, tab_r):
    ub = jnp.concatenate([ugT_r[...], umT_r[...]], axis=0).T
    ib = jnp.concatenate([igT_r[...], imT_r[...]], axis=0).T
    ul = lax.bitcast_convert_type(ub.astype(jnp.bfloat16),
                                  jnp.uint16).astype(jnp.uint32)
    il = lax.bitcast_convert_type(ib.astype(jnp.bfloat16),
                                  jnp.uint16).astype(jnp.uint32)
    tab_r[...] = lax.bitcast_convert_type(ul | (il << 16), jnp.int32)


def _tc_pre(ugT, umT, igT, imT):
    col = lambda i: (0, i)
    row = lambda i: (i, 0)
    return pl.pallas_call(
        _pre_body,
        grid=(pl.cdiv(NROWS, RT),),
        in_specs=[
            pl.BlockSpec((MF, RT), col),
            pl.BlockSpec((HALF, RT), col),
            pl.BlockSpec((MF, RT), col),
            pl.BlockSpec((HALF, RT), col),
        ],
        out_specs=pl.BlockSpec((RT, CW), row),
        out_shape=jax.ShapeDtypeStruct((NROWS, CW), jnp.int32),
    )(ugT, umT, igT, imT)


def _sc_gather_bert(item_idx, bert):
    mesh = plsc.VectorSubcoreMesh(core_axis_name="c", subcore_axis_name="s")
    CH = 64
    NCH = BPW // CH

    @functools.partial(
        pl.kernel, mesh=mesh,
        out_type=jax.ShapeDtypeStruct((B, BD), jnp.float32),
        scratch_types=[
            pltpu.VMEM((BPW,), jnp.int32),
            pltpu.VMEM((CH, BD), jnp.float32),
            pltpu.VMEM((CH, BD), jnp.float32),
            pltpu.SemaphoreType.DMA,
            pltpu.SemaphoreType.DMA,
        ])
    def k(iidx_h, bert_h, obert_h, iidx_v, buf_a, buf_b, sem_a, sem_b):
        wid = lax.axis_index("s") * NC + lax.axis_index("c")
        base = wid * BPW
        pltpu.sync_copy(iidx_h.at[pl.ds(base, BPW)], iidx_v)

        pltpu.async_copy(bert_h.at[iidx_v.at[pl.ds(0, CH)]], buf_a, sem_a)

        @pl.loop(0, NCH // 2)
        def _(ci):
            off_a = 2 * ci * CH
            off_b = off_a + CH
            pltpu.make_async_copy(bert_h.at[iidx_v.at[pl.ds(off_a, CH)]],
                                  buf_a, sem_a).wait()
            pltpu.async_copy(bert_h.at[iidx_v.at[pl.ds(off_b, CH)]],
                             buf_b, sem_b)
            pltpu.sync_copy(buf_a, obert_h.at[pl.ds(base + off_a, CH)])
            pltpu.make_async_copy(bert_h.at[iidx_v.at[pl.ds(off_b, CH)]],
                                  buf_b, sem_b).wait()

            @pl.when(ci < NCH // 2 - 1)
            def _():
                pltpu.async_copy(
                    bert_h.at[iidx_v.at[pl.ds(off_b + CH, CH)]], buf_a, sem_a)

            pltpu.sync_copy(buf_b, obert_h.at[pl.ds(base + off_b, CH)])

    return k(item_idx, bert)


def _sc_gather_tab(idx, tab):
    mesh = plsc.VectorSubcoreMesh(core_axis_name="c", subcore_axis_name="s")
    CH = 128
    NCH = BPW // CH

    @functools.partial(
        pl.kernel, mesh=mesh,
        out_type=jax.ShapeDtypeStruct((B, CW), jnp.int32),
        scratch_types=[
            pltpu.VMEM((BPW,), jnp.int32),
            pltpu.VMEM((CH, CW), jnp.int32),
            pltpu.SemaphoreType.DMA,
        ])
    def k(idx_h, tab_h, o_h, idx_v, buf_v, sem):
        wid = lax.axis_index("s") * NC + lax.axis_index("c")
        base = wid * BPW
        pltpu.sync_copy(idx_h.at[pl.ds(base, BPW)], idx_v)

        @pl.loop(0, NCH)
        def _(ci):
            off = ci * CH
            pltpu.async_copy(tab_h.at[idx_v.at[pl.ds(off, CH)]], buf_v,
                             sem).wait()
            pltpu.sync_copy(buf_v, o_h.at[pl.ds(base + off, CH)])

    return k(idx, tab)


BT = 1024


def _unpack_lo(x):
    lo = lax.bitcast_convert_type(x, jnp.uint32) & jnp.uint32(0xFFFF)
    return lax.bitcast_convert_type(lo.astype(jnp.uint16),
                                    jnp.bfloat16).astype(jnp.float32)


def _unpack_hi(x):
    hi = lax.bitcast_convert_type(x, jnp.uint32) >> 16
    return lax.bitcast_convert_type(hi.astype(jnp.uint16),
                                    jnp.bfloat16).astype(jnp.float32)


def _tc_body(u_r, i_r, bt_r, w1u_r, w1i_r, w1b_r, b1_r, w2a_r, w2b_r, b2_r,
             o_r):
    u = _unpack_lo(u_r[...])
    it = _unpack_hi(i_r[...])
    h = (_DOT(u[:, MF:], w1u_r[...]) + _DOT(it[:, MF:], w1i_r[...])
         + _DOT(bt_r[...], w1b_r[...]) + b1_r[...])
    h = jnp.maximum(h, 0.0)
    g = u[:, :MF] * it[:, :MF]
    o = (jnp.sum(g * w2a_r[...], axis=1, keepdims=True)
         + jnp.sum(h * w2b_r[...], axis=1, keepdims=True) + b2_r[...])
    o_r[...] = o


def _tc_main(ug, ig, bertg, w1u, w1i, w1b, b1, W2, b2):
    b1r = b1.reshape(1, H1)
    w2a = W2[:MF, 0].reshape(1, MF)
    w2b = W2[MF:, 0].reshape(1, H1)
    b2r = b2.reshape(1, 1)

    row = lambda i: (i, 0)
    fixed = lambda i: (0, 0)
    return pl.pallas_call(
        _tc_body,
        grid=(B // BT,),
        in_specs=[
            pl.BlockSpec((BT, CW), row),
            pl.BlockSpec((BT, CW), row),
            pl.BlockSpec((BT, BD), row),
            pl.BlockSpec((HALF, H1), fixed),
            pl.BlockSpec((HALF, H1), fixed),
            pl.BlockSpec((BD, H1), fixed),
            pl.BlockSpec((1, H1), fixed),
            pl.BlockSpec((1, MF), fixed),
            pl.BlockSpec((1, H1), fixed),
            pl.BlockSpec((1, 1), fixed),
        ],
        out_specs=pl.BlockSpec((BT, 1), row),
        out_shape=jax.ShapeDtypeStruct((B, 1), jnp.float32),
    )(ug, ig, bertg, w1u, w1i, w1b, b1r, w2a, w2b, b2r)


def kernel(user_idx, item_idx, user_emb_gmf, item_emb_gmf, user_emb_mlp,
           item_emb_mlp, item_bert, W1, b1, W2, b2):
    user_idx = user_idx.astype(jnp.int32)
    item_idx = item_idx.astype(jnp.int32)
    w1u = W1[:HALF]
    w1i = W1[HALF:2 * HALF]
    w1b = W1[2 * HALF:]
    bertg = _sc_gather_bert(item_idx, item_bert)
    ptab = _tc_pre(user_emb_gmf.T, user_emb_mlp.T, item_emb_gmf.T,
                   item_emb_mlp.T)
    ptab, bertg = lax.optimization_barrier((ptab, bertg))
    utg = _sc_gather_tab(user_idx, ptab)
    itg = _sc_gather_tab(item_idx, ptab)
    return _tc_main(utg, itg, bertg, w1u, w1i, w1b, b1, W2, b2)


# speedup vs baseline: 1.1501x; 1.0387x over previous
import functools

import jax
import jax.numpy as jnp
from jax import lax
from jax.experimental import pallas as pl
from jax.experimental.pallas import tpu as pltpu
from jax.experimental.pallas import tpu_sc as plsc

B = 16384
MF = 64
HALF = 64
BD = 768
H1 = 64
CW = MF + H1
NROWS = 100000

NC = 2
NS = 16
NW = NC * NS
BPW = B // NW

_DOT = functools.partial(jnp.dot, preferred_element_type=jnp.float32,
                         precision=lax.Precision.DEFAULT)


RT = 4096


def _pre_body(ugT_r, umT_r, igT_r, imT_r, tab_r):
    ub = jnp.concatenate([ugT_r[...], umT_r[...]], axis=0).T
    ib = jnp.concatenate([igT_r[...], imT_r[...]], axis=0).T
    ul = lax.bitcast_convert_type(ub.astype(jnp.bfloat16),
                                  jnp.uint16).astype(jnp.uint32)
    il = lax.bitcast_convert_type(ib.astype(jnp.bfloat16),
                                  jnp.uint16).astype(jnp.uint32)
    tab_r[...] = lax.bitcast_convert_type(ul | (il << 16), jnp.int32)


def _tc_pre(ugT, umT, igT, imT):
    col = lambda i: (0, i)
    row = lambda i: (i, 0)
    return pl.pallas_call(
        _pre_body,
        grid=(pl.cdiv(NROWS, RT),),
        in_specs=[
            pl.BlockSpec((MF, RT), col),
            pl.BlockSpec((HALF, RT), col),
            pl.BlockSpec((MF, RT), col),
            pl.BlockSpec((HALF, RT), col),
        ],
        out_specs=pl.BlockSpec((RT, CW), row),
        out_shape=jax.ShapeDtypeStruct((NROWS, CW), jnp.int32),
    )(ugT, umT, igT, imT)


def _sc_gather_bert(item_idx, bert):
    mesh = plsc.VectorSubcoreMesh(core_axis_name="c", subcore_axis_name="s")
    CH = 64
    NCH = BPW // CH

    @functools.partial(
        pl.kernel, mesh=mesh,
        out_type=jax.ShapeDtypeStruct((B, BD), jnp.float32),
        scratch_types=[
            pltpu.VMEM((BPW,), jnp.int32),
            pltpu.VMEM((CH, BD), jnp.float32),
            pltpu.VMEM((CH, BD), jnp.float32),
            pltpu.SemaphoreType.DMA,
            pltpu.SemaphoreType.DMA,
        ])
    def k(iidx_h, bert_h, obert_h, iidx_v, buf_a, buf_b, sem_a, sem_b):
        wid = lax.axis_index("s") * NC + lax.axis_index("c")
        base = wid * BPW
        pltpu.sync_copy(iidx_h.at[pl.ds(base, BPW)], iidx_v)

        pltpu.async_copy(bert_h.at[iidx_v.at[pl.ds(0, CH)]], buf_a, sem_a)

        @pl.loop(0, NCH // 2)
        def _(ci):
            off_a = 2 * ci * CH
            off_b = off_a + CH
            pltpu.make_async_copy(bert_h.at[iidx_v.at[pl.ds(off_a, CH)]],
                                  buf_a, sem_a).wait()
            pltpu.async_copy(bert_h.at[iidx_v.at[pl.ds(off_b, CH)]],
                             buf_b, sem_b)
            pltpu.sync_copy(buf_a, obert_h.at[pl.ds(base + off_a, CH)])
            pltpu.make_async_copy(bert_h.at[iidx_v.at[pl.ds(off_b, CH)]],
                                  buf_b, sem_b).wait()

            @pl.when(ci < NCH // 2 - 1)
            def _():
                pltpu.async_copy(
                    bert_h.at[iidx_v.at[pl.ds(off_b + CH, CH)]], buf_a, sem_a)

            pltpu.sync_copy(buf_b, obert_h.at[pl.ds(base + off_b, CH)])

    return k(item_idx, bert)


def _sc_gather_ui(user_idx, item_idx, tab):
    mesh = plsc.VectorSubcoreMesh(core_axis_name="c", subcore_axis_name="s")
    CH = 128
    NCH = BPW // CH

    @functools.partial(
        pl.kernel, mesh=mesh,
        out_type=[jax.ShapeDtypeStruct((B, CW), jnp.int32),
                  jax.ShapeDtypeStruct((B, CW), jnp.int32)],
        scratch_types=[
            pltpu.VMEM((BPW,), jnp.int32),
            pltpu.VMEM((BPW,), jnp.int32),
            pltpu.VMEM((CH, CW), jnp.int32),
            pltpu.VMEM((CH, CW), jnp.int32),
            pltpu.SemaphoreType.DMA,
            pltpu.SemaphoreType.DMA,
        ])
    def k(uidx_h, iidx_h, tab_h, ou_h, oi_h,
          uidx_v, iidx_v, bu_v, bi_v, sem_u, sem_i):
        wid = lax.axis_index("s") * NC + lax.axis_index("c")
        base = wid * BPW
        pltpu.sync_copy(uidx_h.at[pl.ds(base, BPW)], uidx_v)
        pltpu.sync_copy(iidx_h.at[pl.ds(base, BPW)], iidx_v)

        @pl.loop(0, NCH)
        def _(ci):
            off = ci * CH
            cu = pltpu.async_copy(tab_h.at[uidx_v.at[pl.ds(off, CH)]],
                                  bu_v, sem_u)
            cit = pltpu.async_copy(tab_h.at[iidx_v.at[pl.ds(off, CH)]],
                                   bi_v, sem_i)
            cu.wait()
            cit.wait()
            dst = pl.ds(base + off, CH)
            pltpu.sync_copy(bu_v, ou_h.at[dst])
            pltpu.sync_copy(bi_v, oi_h.at[dst])

    return k(user_idx, item_idx, tab)


BT = 1024


def _unpack_lo(x):
    lo = lax.bitcast_convert_type(x, jnp.uint32) & jnp.uint32(0xFFFF)
    return lax.bitcast_convert_type(lo.astype(jnp.uint16),
                                    jnp.bfloat16).astype(jnp.float32)


def _unpack_hi(x):
    hi = lax.bitcast_convert_type(x, jnp.uint32) >> 16
    return lax.bitcast_convert_type(hi.astype(jnp.uint16),
                                    jnp.bfloat16).astype(jnp.float32)


def _tc_body(u_r, i_r, bt_r, w1u_r, w1i_r, w1b_r, b1_r, w2a_r, w2b_r, b2_r,
             o_r):
    u = _unpack_lo(u_r[...])
    it = _unpack_hi(i_r[...])
    h = (_DOT(u[:, MF:], w1u_r[...]) + _DOT(it[:, MF:], w1i_r[...])
         + _DOT(bt_r[...], w1b_r[...]) + b1_r[...])
    h = jnp.maximum(h, 0.0)
    g = u[:, :MF] * it[:, :MF]
    o = (jnp.sum(g * w2a_r[...], axis=1, keepdims=True)
         + jnp.sum(h * w2b_r[...], axis=1, keepdims=True) + b2_r[...])
    o_r[...] = o


def _tc_main(ug, ig, bertg, w1u, w1i, w1b, b1, W2, b2):
    b1r = b1.reshape(1, H1)
    w2a = W2[:MF, 0].reshape(1, MF)
    w2b = W2[MF:, 0].reshape(1, H1)
    b2r = b2.reshape(1, 1)

    row = lambda i: (i, 0)
    fixed = lambda i: (0, 0)
    return pl.pallas_call(
        _tc_body,
        grid=(B // BT,),
        in_specs=[
            pl.BlockSpec((BT, CW), row),
            pl.BlockSpec((BT, CW), row),
            pl.BlockSpec((BT, BD), row),
            pl.BlockSpec((HALF, H1), fixed),
            pl.BlockSpec((HALF, H1), fixed),
            pl.BlockSpec((BD, H1), fixed),
            pl.BlockSpec((1, H1), fixed),
            pl.BlockSpec((1, MF), fixed),
            pl.BlockSpec((1, H1), fixed),
            pl.BlockSpec((1, 1), fixed),
        ],
        out_specs=pl.BlockSpec((BT, 1), row),
        out_shape=jax.ShapeDtypeStruct((B, 1), jnp.float32),
    )(ug, ig, bertg, w1u, w1i, w1b, b1r, w2a, w2b, b2r)


def kernel(user_idx, item_idx, user_emb_gmf, item_emb_gmf, user_emb_mlp,
           item_emb_mlp, item_bert, W1, b1, W2, b2):
    user_idx = user_idx.astype(jnp.int32)
    item_idx = item_idx.astype(jnp.int32)
    w1u = W1[:HALF]
    w1i = W1[HALF:2 * HALF]
    w1b = W1[2 * HALF:]
    bertg = _sc_gather_bert(item_idx, item_bert)
    ptab = _tc_pre(user_emb_gmf.T, user_emb_mlp.T, item_emb_gmf.T,
                   item_emb_mlp.T)
    ptab, bertg = lax.optimization_barrier((ptab, bertg))
    utg, itg = _sc_gather_ui(user_idx, item_idx, ptab)
    return _tc_main(utg, itg, bertg, w1u, w1i, w1b, b1, W2, b2)
